# Initial kernel scaffold; baseline (speedup 1.0000x reference)
import jax
import jax.numpy as jnp
from jax.experimental import pallas as pl


def kernel(x, pad_hv, idx_identit, idx_out, w1, w2, w3, b, hout, wout):
    raise NotImplementedError("write your pallas kernel here")


# speedup vs baseline: 15.2536x; 15.2536x over previous
import math

import jax
import jax.numpy as jnp
from jax import lax
from jax.experimental import pallas as pl
from jax.experimental.pallas import tpu as pltpu
from jax.experimental.pallas import tpu_sc as plsc

_BIG_K = 13
_SMALL_K = 3
_NK = math.ceil(_BIG_K / _SMALL_K)
_PAD = _SMALL_K - 1
_MID = _BIG_K // 2
_SHIFTS = tuple(_MID - i * _SMALL_K - _PAD for i in range(_NK))
_EXTRA = _PAD - _SMALL_K // 2

_L = 16
_HIN = 58
_WIN = 58
_H = 56
_W = 56
_ROWP = 64
_CROW = _HIN * _WIN
_XW = _NK * _CROW
_XWP = _XW + 16
_OW = _H * _W
_G = 4
_CIN = 480
_COUT = 96
_JB_IN = (0, 16, 32, 42)
_JB_OUT = (0, 16, 32, 40)
_NW = 32


def _iota():
    return lax.iota(jnp.int32, _L)


def _splat_lane(vec, lane):
    return jnp.take_along_axis(
        vec, jnp.full((_L,), lane, jnp.int32), axis=0,
        mode=lax.GatherScatterMode.PROMISE_IN_BOUNDS)


def _col_mask(s, jbase):
    q = 1 + _iota() + jbase + s
    return (q >= 0) & (q < _WIN)


def _body(x_hbm, ph_hbm, w1_hbm, w2_hbm, id_hbm, w3_hbm,
          o1_hbm, o2_hbm, o3_hbm,
          xbuf0, xbuf1, ybuf, zbuf, ubuf, ob1, ob2, ob3,
          phv, w1v, w2v, w3v, idv, ws1, ws2, ws3,
          dsem0, dsem1):
    nunits = o1_hbm.shape[0]
    wid = lax.axis_index("s") * 2 + lax.axis_index("c")
    per_w = nunits // _NW

    pltpu.sync_copy(ph_hbm, phv)
    pltpu.sync_copy(w1_hbm, w1v)
    pltpu.sync_copy(w2_hbm, w2v)
    pltpu.sync_copy(w3_hbm, w3v)
    pltpu.sync_copy(id_hbm, idv)

    for cv in range(_CIN // _L):
        c0 = cv * _L
        for si, s in enumerate(_SHIFTS):
            acc1 = jnp.zeros((_L,), jnp.float32)
            acc2 = jnp.zeros((_L,), jnp.float32)
            for g in range(_G):
                ph1 = phv[pl.ds(g * _CIN + c0, _L)]
                ph2 = phv[pl.ds((_G + g) * _CIN + c0, _L)]
                wa = w1v[pl.ds(g * _CIN + c0, _L)]
                wb = w2v[pl.ds(g * _CIN + c0, _L)]
                acc1 = acc1 + jnp.where(ph1 == s, wa, 0.0)
                acc2 = acc2 + jnp.where(ph2 == s, wb, 0.0)
            ws1[pl.ds(si * _CIN + c0, _L)] = acc1
            ws2[pl.ds(si * _CIN + c0, _L)] = acc2
    for cov in range(_COUT // _L):
        co0 = cov * _L
        covec = jnp.full((_L,), co0, jnp.int32) + _iota()
        for k in range(_NK):
            acc3 = jnp.zeros((_L,), jnp.float32)
            for g in range(_G):
                idg = idv[pl.ds(g * _COUT + co0, _L)]
                wg = w3v[pl.ds(g * _COUT + co0, _L)]
                acc3 = acc3 + jnp.where(idg == covec * _NK + k, wg, 0.0)
            ws3[pl.ds(k * _COUT + co0, _L)] = acc3

    def unit_of(ul):
        return wid * per_w + ul

    pltpu.make_async_copy(x_hbm.at[unit_of(0)],
                          xbuf0.at[pl.ds(0, _XW)], dsem0).start()

    def unit_step(ul):
        u = unit_of(ul)

        @pl.when(ul % 2 == 0)
        def _():
            pltpu.make_async_copy(x_hbm.at[u], xbuf0.at[pl.ds(0, _XW)], dsem0).wait()

        @pl.when(ul % 2 == 1)
        def _():
            pltpu.make_async_copy(x_hbm.at[u], xbuf1.at[pl.ds(0, _XW)], dsem1).wait()

        @pl.when((ul + 1 < per_w) & (ul % 2 == 0))
        def _():
            pltpu.make_async_copy(x_hbm.at[unit_of(ul + 1)],
                                  xbuf1.at[pl.ds(0, _XW)], dsem1).start()

        @pl.when((ul + 1 < per_w) & (ul % 2 == 1))
        def _():
            pltpu.make_async_copy(x_hbm.at[unit_of(ul + 1)],
                                  xbuf0.at[pl.ds(0, _XW)], dsem0).start()

        co = u % _COUT
        c0 = co * _NK

        wy = [[None] * _NK for _ in range(_NK)]
        wz = [[None] * _NK for _ in range(_NK)]
        wu = [None] * _NK
        for si in range(_NK):
            v1 = ws1[pl.ds(si * _CIN + c0, _L)]
            v2 = ws2[pl.ds(si * _CIN + c0, _L)]
            for cl in range(_NK):
                wy[cl][si] = _splat_lane(v1, cl)
                wz[cl][si] = _splat_lane(v2, cl)
        lane3 = co % _L
        for cl in range(_NK):
            v3 = ws3[pl.ds(cl * _COUT + (co // _L) * _L, _L)]
            wu[cl] = _splat_lane(v3, lane3)

        def compute(xref):
            def rowA1(r):
                for jb in _JB_IN:
                    xs = [xref[pl.ds(cl * _CROW + r * _WIN + jb, _L)]
                          for cl in range(_NK)]
                    uacc = xs[0] * wu[0]
                    for cl in range(1, _NK):
                        uacc = uacc + xs[cl] * wu[cl]
                    ubuf[pl.ds(r * _ROWP + jb, _L)] = uacc
                    for si in range(_NK):
                        yacc = xs[0] * wy[0][si]
                        for cl in range(1, _NK):
                            yacc = yacc + xs[cl] * wy[cl][si]
                        ybuf[pl.ds(si * (_HIN * _ROWP) + r * _ROWP + jb, _L)] = yacc

            pl.loop(0, _HIN)(rowA1)

            def rowA2(r):
                for jb in _JB_IN:
                    xs = [xref[pl.ds(cl * _CROW + r * _WIN + jb, _L)]
                          for cl in range(_NK)]
                    for si in range(_NK):
                        zacc = xs[0] * wz[0][si]
                        for cl in range(1, _NK):
                            zacc = zacc + xs[cl] * wz[cl][si]
                        zbuf[pl.ds(si * (_HIN * _ROWP) + r * _ROWP + jb, _L)] = zacc

            pl.loop(0, _HIN)(rowA2)

            def rowB(i):
                r = i + 1
                for jb in _JB_OUT:
                    a1 = jnp.zeros((_L,), jnp.float32)
                    for si, s in enumerate(_SHIFTS):
                        v = ybuf[pl.ds(si * (_HIN * _ROWP) + r * _ROWP
                                       + 1 + jb + s, _L)]
                        a1 = a1 + jnp.where(_col_mask(s, jb), v, 0.0)
                    a2 = jnp.zeros((_L,), jnp.float32)
                    for si, s in enumerate(_SHIFTS):
                        rr = r + s
                        ok = (rr >= 0) & (rr < _HIN)
                        rrc = jnp.clip(rr, 0, _HIN - 1)
                        v = zbuf[pl.ds(si * (_HIN * _ROWP) + rrc * _ROWP
                                       + 1 + jb, _L)]
                        a2 = a2 + jnp.where(ok, v, 0.0)
                    a3 = ubuf[pl.ds(r * _ROWP + 1 + jb, _L)]
                    ob1[pl.ds(i * _W + jb, _L)] = a1
                    ob2[pl.ds(i * _W + jb, _L)] = a2
                    ob3[pl.ds(i * _W + jb, _L)] = a3

            pl.loop(0, _H)(rowB)

        @pl.when(ul % 2 == 0)
        def _():
            compute(xbuf0)

        @pl.when(ul % 2 == 1)
        def _():
            compute(xbuf1)

        pltpu.sync_copy(ob1, o1_hbm.at[u])
        pltpu.sync_copy(ob2, o2_hbm.at[u])
        pltpu.sync_copy(ob3, o3_hbm.at[u])

    pl.loop(0, per_w)(unit_step)


def _sc_call(x2, ph_t, w1, w2, id_t, w3):
    nunits = x2.shape[0]
    mesh = plsc.VectorSubcoreMesh(core_axis_name="c", subcore_axis_name="s",
                                  num_cores=2, num_subcores=16)
    oshape = jax.ShapeDtypeStruct((nunits, _OW), jnp.float32)
    kfn = pl.kernel(
        _body,
        mesh=mesh,
        compiler_params=pltpu.CompilerParams(use_tc_tiling_on_sc=False),
        out_type=[oshape, oshape, oshape],
        scratch_types=[
            pltpu.VMEM((_XWP,), jnp.float32),
            pltpu.VMEM((_XWP,), jnp.float32),
            pltpu.VMEM((_NK * _HIN * _ROWP,), jnp.float32),
            pltpu.VMEM((_NK * _HIN * _ROWP,), jnp.float32),
            pltpu.VMEM((_HIN * _ROWP,), jnp.float32),
            pltpu.VMEM((_OW,), jnp.float32),
            pltpu.VMEM((_OW,), jnp.float32),
            pltpu.VMEM((_OW,), jnp.float32),
            pltpu.VMEM((2 * _G * _CIN,), jnp.int32),
            pltpu.VMEM((_G * _CIN,), jnp.float32),
            pltpu.VMEM((_G * _CIN,), jnp.float32),
            pltpu.VMEM((_G * _COUT,), jnp.float32),
            pltpu.VMEM((_G * _COUT,), jnp.int32),
            pltpu.VMEM((_NK * _CIN,), jnp.float32),
            pltpu.VMEM((_NK * _CIN,), jnp.float32),
            pltpu.VMEM((_NK * _COUT,), jnp.float32),
            pltpu.SemaphoreType.DMA,
            pltpu.SemaphoreType.DMA,
        ],
    )
    return kfn(x2, ph_t, w1, w2, id_t, w3)


def kernel(x, pad_hv, idx_identit, idx_out, w1, w2, w3, b, hout, wout):
    B, c_in, Hin, Win = x.shape
    c_out = idx_identit.shape[0]
    x2 = x.reshape(B * c_out, _NK * Hin * Win)
    o1, o2, o3 = _sc_call(x2, pad_hv.T.reshape(-1), w1, w2,
                          idx_identit.T.reshape(-1), w3)
    H, W = Hin - 2 * _EXTRA, Win - 2 * _EXTRA
    return (o1.reshape(B, c_out, H, W),
            o2.reshape(B, c_out, H, W),
            o3.reshape(B, c_out, H, W))


# speedup vs baseline: 15.8896x; 1.0417x over previous
import math

import jax
import jax.numpy as jnp
from jax import lax
from jax.experimental import pallas as pl
from jax.experimental.pallas import tpu as pltpu
from jax.experimental.pallas import tpu_sc as plsc

_BIG_K = 13
_SMALL_K = 3
_NK = math.ceil(_BIG_K / _SMALL_K)
_PAD = _SMALL_K - 1
_MID = _BIG_K // 2
_SHIFTS = tuple(_MID - i * _SMALL_K - _PAD for i in range(_NK))
_EXTRA = _PAD - _SMALL_K // 2

_L = 16
_HIN = 58
_WIN = 58
_H = 56
_W = 56
_ROWP = 64
_CROW = _HIN * _WIN
_XW = _NK * _CROW
_XB = _XW + 20
_OW = _H * _W
_G = 4
_CIN = 480
_COUT = 96
_JB_IN = (0, 16, 32, 42)
_JB_OUT = (0, 16, 32, 40)
_NW = 32


def _iota():
    return lax.iota(jnp.int32, _L)


def _splat_lane(vec, lane):
    return jnp.take_along_axis(
        vec, jnp.full((_L,), lane, jnp.int32), axis=0,
        mode=lax.GatherScatterMode.PROMISE_IN_BOUNDS)


def _col_mask(s, jbase):
    q = 1 + _iota() + jbase + s
    return (q >= 0) & (q < _WIN)


def _body(x_hbm, ph_hbm, w1_hbm, w2_hbm, id_hbm, w3_hbm,
          o1_hbm, o2_hbm, o3_hbm,
          xbuf, ybuf, zbuf, ubuf, ob1, ob2, ob3,
          phv, w1v, w2v, w3v, idv, ws1, ws2, ws3,
          dsem0, dsem1):
    nunits = o1_hbm.shape[0]
    wid = lax.axis_index("s") * 2 + lax.axis_index("c")
    per_w = nunits // _NW

    pltpu.sync_copy(ph_hbm, phv)
    pltpu.sync_copy(w1_hbm, w1v)
    pltpu.sync_copy(w2_hbm, w2v)
    pltpu.sync_copy(w3_hbm, w3v)
    pltpu.sync_copy(id_hbm, idv)

    for cv in range(_CIN // _L):
        c0 = cv * _L
        for si, s in enumerate(_SHIFTS):
            acc1 = jnp.zeros((_L,), jnp.float32)
            acc2 = jnp.zeros((_L,), jnp.float32)
            for g in range(_G):
                ph1 = phv[pl.ds(g * _CIN + c0, _L)]
                ph2 = phv[pl.ds((_G + g) * _CIN + c0, _L)]
                wa = w1v[pl.ds(g * _CIN + c0, _L)]
                wb = w2v[pl.ds(g * _CIN + c0, _L)]
                acc1 = acc1 + jnp.where(ph1 == s, wa, 0.0)
                acc2 = acc2 + jnp.where(ph2 == s, wb, 0.0)
            ws1[pl.ds(si * _CIN + c0, _L)] = acc1
            ws2[pl.ds(si * _CIN + c0, _L)] = acc2
    for cov in range(_COUT // _L):
        co0 = cov * _L
        covec = jnp.full((_L,), co0, jnp.int32) + _iota()
        for k in range(_NK):
            acc3 = jnp.zeros((_L,), jnp.float32)
            for g in range(_G):
                idg = idv[pl.ds(g * _COUT + co0, _L)]
                wg = w3v[pl.ds(g * _COUT + co0, _L)]
                acc3 = acc3 + jnp.where(idg == covec * _NK + k, wg, 0.0)
            ws3[pl.ds(k * _COUT + co0, _L)] = acc3

    def unit_of(ul):
        return wid * per_w + ul

    pltpu.make_async_copy(x_hbm.at[unit_of(0)],
                          xbuf.at[pl.ds(0, _XW)], dsem0).start()

    def unit_step(ul):
        u = unit_of(ul)
        base = (ul % 2) * _XB

        @pl.when(ul % 2 == 0)
        def _():
            pltpu.make_async_copy(x_hbm.at[u], xbuf.at[pl.ds(0, _XW)], dsem0).wait()

        @pl.when(ul % 2 == 1)
        def _():
            pltpu.make_async_copy(x_hbm.at[u], xbuf.at[pl.ds(_XB, _XW)], dsem1).wait()

        @pl.when((ul + 1 < per_w) & (ul % 2 == 0))
        def _():
            pltpu.make_async_copy(x_hbm.at[unit_of(ul + 1)],
                                  xbuf.at[pl.ds(_XB, _XW)], dsem1).start()

        @pl.when((ul + 1 < per_w) & (ul % 2 == 1))
        def _():
            pltpu.make_async_copy(x_hbm.at[unit_of(ul + 1)],
                                  xbuf.at[pl.ds(0, _XW)], dsem0).start()

        co = u % _COUT
        c0 = co * _NK

        wy = [[None] * _NK for _ in range(_NK)]
        wz = [[None] * _NK for _ in range(_NK)]
        wu = [None] * _NK
        for si in range(_NK):
            v1 = ws1[pl.ds(si * _CIN + c0, _L)]
            v2 = ws2[pl.ds(si * _CIN + c0, _L)]
            for cl in range(_NK):
                wy[cl][si] = _splat_lane(v1, cl)
                wz[cl][si] = _splat_lane(v2, cl)
        lane3 = co % _L
        for cl in range(_NK):
            v3 = ws3[pl.ds(cl * _COUT + (co // _L) * _L, _L)]
            wu[cl] = _splat_lane(v3, lane3)

        def compute(xref):
            def rowA1(r):
                for jb in _JB_IN:
                    xs = [xref[pl.ds(base + cl * _CROW + r * _WIN + jb, _L)]
                          for cl in range(_NK)]
                    uacc = xs[0] * wu[0]
                    for cl in range(1, _NK):
                        uacc = uacc + xs[cl] * wu[cl]
                    ubuf[pl.ds(r * _ROWP + jb, _L)] = uacc
                    for si in range(_NK):
                        yacc = xs[0] * wy[0][si]
                        for cl in range(1, _NK):
                            yacc = yacc + xs[cl] * wy[cl][si]
                        ybuf[pl.ds(si * (_HIN * _ROWP) + r * _ROWP + jb, _L)] = yacc

            pl.loop(0, _HIN, unroll=4)(rowA1)

            def rowA2(r):
                for jb in _JB_IN:
                    xs = [xref[pl.ds(base + cl * _CROW + r * _WIN + jb, _L)]
                          for cl in range(_NK)]
                    for si in range(_NK):
                        zacc = xs[0] * wz[0][si]
                        for cl in range(1, _NK):
                            zacc = zacc + xs[cl] * wz[cl][si]
                        zbuf[pl.ds(si * (_HIN * _ROWP) + r * _ROWP + jb, _L)] = zacc

            pl.loop(0, _HIN, unroll=4)(rowA2)

            def rowB(i):
                r = i + 1
                for jb in _JB_OUT:
                    a1 = jnp.zeros((_L,), jnp.float32)
                    for si, s in enumerate(_SHIFTS):
                        v = ybuf[pl.ds(si * (_HIN * _ROWP) + r * _ROWP
                                       + 1 + jb + s, _L)]
                        a1 = a1 + jnp.where(_col_mask(s, jb), v, 0.0)
                    a2 = jnp.zeros((_L,), jnp.float32)
                    for si, s in enumerate(_SHIFTS):
                        rr = r + s
                        ok = (rr >= 0) & (rr < _HIN)
                        rrc = jnp.clip(rr, 0, _HIN - 1)
                        v = zbuf[pl.ds(si * (_HIN * _ROWP) + rrc * _ROWP
                                       + 1 + jb, _L)]
                        a2 = a2 + jnp.where(ok, v, 0.0)
                    a3 = ubuf[pl.ds(r * _ROWP + 1 + jb, _L)]
                    ob1[pl.ds(i * _W + jb, _L)] = a1
                    ob2[pl.ds(i * _W + jb, _L)] = a2
                    ob3[pl.ds(i * _W + jb, _L)] = a3

            pl.loop(0, _H, unroll=2)(rowB)

        compute(xbuf)

        pltpu.sync_copy(ob1, o1_hbm.at[u])
        pltpu.sync_copy(ob2, o2_hbm.at[u])
        pltpu.sync_copy(ob3, o3_hbm.at[u])

    pl.loop(0, per_w)(unit_step)


def _sc_call(x2, ph_t, w1, w2, id_t, w3):
    nunits = x2.shape[0]
    mesh = plsc.VectorSubcoreMesh(core_axis_name="c", subcore_axis_name="s",
                                  num_cores=2, num_subcores=16)
    oshape = jax.ShapeDtypeStruct((nunits, _OW), jnp.float32)
    kfn = pl.kernel(
        _body,
        mesh=mesh,
        compiler_params=pltpu.CompilerParams(use_tc_tiling_on_sc=False),
        out_type=[oshape, oshape, oshape],
        scratch_types=[
            pltpu.VMEM((2 * _XB,), jnp.float32),
            pltpu.VMEM((_NK * _HIN * _ROWP,), jnp.float32),
            pltpu.VMEM((_NK * _HIN * _ROWP,), jnp.float32),
            pltpu.VMEM((_HIN * _ROWP,), jnp.float32),
            pltpu.VMEM((_OW,), jnp.float32),
            pltpu.VMEM((_OW,), jnp.float32),
            pltpu.VMEM((_OW,), jnp.float32),
            pltpu.VMEM((2 * _G * _CIN,), jnp.int32),
            pltpu.VMEM((_G * _CIN,), jnp.float32),
            pltpu.VMEM((_G * _CIN,), jnp.float32),
            pltpu.VMEM((_G * _COUT,), jnp.float32),
            pltpu.VMEM((_G * _COUT,), jnp.int32),
            pltpu.VMEM((_NK * _CIN,), jnp.float32),
            pltpu.VMEM((_NK * _CIN,), jnp.float32),
            pltpu.VMEM((_NK * _COUT,), jnp.float32),
            pltpu.SemaphoreType.DMA,
            pltpu.SemaphoreType.DMA,
        ],
    )
    return kfn(x2, ph_t, w1, w2, id_t, w3)


def kernel(x, pad_hv, idx_identit, idx_out, w1, w2, w3, b, hout, wout):
    B, c_in, Hin, Win = x.shape
    c_out = idx_identit.shape[0]
    x2 = x.reshape(B * c_out, _NK * Hin * Win)
    o1, o2, o3 = _sc_call(x2, pad_hv.T.reshape(-1), w1, w2,
                          idx_identit.T.reshape(-1), w3)
    H, W = Hin - 2 * _EXTRA, Win - 2 * _EXTRA
    return (o1.reshape(B, c_out, H, W),
            o2.reshape(B, c_out, H, W),
            o3.reshape(B, c_out, H, W))


# speedup vs baseline: 16.2134x; 1.0204x over previous
import math

import jax
import jax.numpy as jnp
from jax import lax
from jax.experimental import pallas as pl
from jax.experimental.pallas import tpu as pltpu
from jax.experimental.pallas import tpu_sc as plsc

_BIG_K = 13
_SMALL_K = 3
_NK = math.ceil(_BIG_K / _SMALL_K)
_PAD = _SMALL_K - 1
_MID = _BIG_K // 2
_SHIFTS = tuple(_MID - i * _SMALL_K - _PAD for i in range(_NK))
_EXTRA = _PAD - _SMALL_K // 2

_L = 16
_HIN = 58
_WIN = 58
_H = 56
_W = 56
_ROWP = 64
_CROW = _HIN * _WIN
_XW = _NK * _CROW
_XB = _XW + 20
_OW = _H * _W
_G = 4
_CIN = 480
_COUT = 96
_JB_IN = (0, 16, 32, 42)
_JB_OUT = (0, 16, 32, 40)
_NW = 32


def _iota():
    return lax.iota(jnp.int32, _L)


def _splat_lane(vec, lane):
    return jnp.take_along_axis(
        vec, jnp.full((_L,), lane, jnp.int32), axis=0,
        mode=lax.GatherScatterMode.PROMISE_IN_BOUNDS)


def _col_mask(s, jbase):
    q = 1 + _iota() + jbase + s
    return (q >= 0) & (q < _WIN)


def _body(x_hbm, ph_hbm, w1_hbm, w2_hbm, id_hbm, w3_hbm,
          o1_hbm, o2_hbm, o3_hbm,
          xbuf, ybuf, zbuf, ubuf, ob1, ob2, ob3,
          ws1, ws2, ws3,
          dsem0, dsem1, osem0, osem1):
    nunits = o1_hbm.shape[0]
    wid = lax.axis_index("s") * 2 + lax.axis_index("c")
    per_w = nunits // _NW

    _PH, _W1, _W2, _W3, _ID = 0, 3840, 5760, 7680, 8064
    pltpu.sync_copy(ph_hbm, zbuf.at[pl.ds(_PH, 2 * _G * _CIN)])
    pltpu.sync_copy(w1_hbm, zbuf.at[pl.ds(_W1, _G * _CIN)])
    pltpu.sync_copy(w2_hbm, zbuf.at[pl.ds(_W2, _G * _CIN)])
    pltpu.sync_copy(w3_hbm, zbuf.at[pl.ds(_W3, _G * _COUT)])
    pltpu.sync_copy(id_hbm, zbuf.at[pl.ds(_ID, _G * _COUT)])

    for cv in range(_CIN // _L):
        c0 = cv * _L
        for si, s in enumerate(_SHIFTS):
            acc1 = jnp.zeros((_L,), jnp.float32)
            acc2 = jnp.zeros((_L,), jnp.float32)
            for g in range(_G):
                ph1 = zbuf[pl.ds(_PH + g * _CIN + c0, _L)]
                ph2 = zbuf[pl.ds(_PH + (_G + g) * _CIN + c0, _L)]
                wa = zbuf[pl.ds(_W1 + g * _CIN + c0, _L)]
                wb = zbuf[pl.ds(_W2 + g * _CIN + c0, _L)]
                acc1 = acc1 + jnp.where(ph1 == float(s), wa, 0.0)
                acc2 = acc2 + jnp.where(ph2 == float(s), wb, 0.0)
            ws1[pl.ds(si * _CIN + c0, _L)] = acc1
            ws2[pl.ds(si * _CIN + c0, _L)] = acc2
    for cov in range(_COUT // _L):
        co0 = cov * _L
        covec = jnp.full((_L,), co0, jnp.int32) + _iota()
        for k in range(_NK):
            acc3 = jnp.zeros((_L,), jnp.float32)
            for g in range(_G):
                idg = zbuf[pl.ds(_ID + g * _COUT + co0, _L)]
                wg = zbuf[pl.ds(_W3 + g * _COUT + co0, _L)]
                acc3 = acc3 + jnp.where(idg == (covec * _NK + k).astype(jnp.float32),
                                        wg, 0.0)
            ws3[pl.ds(k * _COUT + co0, _L)] = acc3

    def unit_of(ul):
        return wid * per_w + ul

    pltpu.make_async_copy(x_hbm.at[unit_of(0)],
                          xbuf.at[pl.ds(0, _XW)], dsem0).start()

    def unit_step(ul):
        u = unit_of(ul)
        base = (ul % 2) * _XB
        base2 = (ul % 2) * _OW

        @pl.when((ul >= 2) & (ul % 2 == 0))
        def _():
            pltpu.make_async_copy(ob1.at[pl.ds(base2, _OW)], o1_hbm.at[u - 2], osem0).wait()
            pltpu.make_async_copy(ob2.at[pl.ds(base2, _OW)], o2_hbm.at[u - 2], osem0).wait()
            pltpu.make_async_copy(ob3.at[pl.ds(base2, _OW)], o3_hbm.at[u - 2], osem0).wait()

        @pl.when((ul >= 2) & (ul % 2 == 1))
        def _():
            pltpu.make_async_copy(ob1.at[pl.ds(base2, _OW)], o1_hbm.at[u - 2], osem1).wait()
            pltpu.make_async_copy(ob2.at[pl.ds(base2, _OW)], o2_hbm.at[u - 2], osem1).wait()
            pltpu.make_async_copy(ob3.at[pl.ds(base2, _OW)], o3_hbm.at[u - 2], osem1).wait()

        @pl.when(ul % 2 == 0)
        def _():
            pltpu.make_async_copy(x_hbm.at[u], xbuf.at[pl.ds(0, _XW)], dsem0).wait()

        @pl.when(ul % 2 == 1)
        def _():
            pltpu.make_async_copy(x_hbm.at[u], xbuf.at[pl.ds(_XB, _XW)], dsem1).wait()

        @pl.when((ul + 1 < per_w) & (ul % 2 == 0))
        def _():
            pltpu.make_async_copy(x_hbm.at[unit_of(ul + 1)],
                                  xbuf.at[pl.ds(_XB, _XW)], dsem1).start()

        @pl.when((ul + 1 < per_w) & (ul % 2 == 1))
        def _():
            pltpu.make_async_copy(x_hbm.at[unit_of(ul + 1)],
                                  xbuf.at[pl.ds(0, _XW)], dsem0).start()

        co = u % _COUT
        c0 = co * _NK

        wy = [[None] * _NK for _ in range(_NK)]
        wz = [[None] * _NK for _ in range(_NK)]
        wu = [None] * _NK
        for si in range(_NK):
            v1 = ws1[pl.ds(si * _CIN + c0, _L)]
            v2 = ws2[pl.ds(si * _CIN + c0, _L)]
            for cl in range(_NK):
                wy[cl][si] = _splat_lane(v1, cl)
                wz[cl][si] = _splat_lane(v2, cl)
        lane3 = co % _L
        for cl in range(_NK):
            v3 = ws3[pl.ds(cl * _COUT + (co // _L) * _L, _L)]
            wu[cl] = _splat_lane(v3, lane3)

        def compute(xref):
            def rowA1(r):
                for jb in _JB_IN:
                    xs = [xref[pl.ds(base + cl * _CROW + r * _WIN + jb, _L)]
                          for cl in range(_NK)]
                    uacc = xs[0] * wu[0]
                    for cl in range(1, _NK):
                        uacc = uacc + xs[cl] * wu[cl]
                    ubuf[pl.ds(r * _ROWP + jb, _L)] = uacc
                    for si in range(_NK):
                        yacc = xs[0] * wy[0][si]
                        for cl in range(1, _NK):
                            yacc = yacc + xs[cl] * wy[cl][si]
                        ybuf[pl.ds(si * (_HIN * _ROWP) + r * _ROWP + jb, _L)] = yacc

            pl.loop(0, _HIN, unroll=2)(rowA1)

            def rowA2(r):
                for jb in _JB_IN:
                    xs = [xref[pl.ds(base + cl * _CROW + r * _WIN + jb, _L)]
                          for cl in range(_NK)]
                    for si in range(_NK):
                        zacc = xs[0] * wz[0][si]
                        for cl in range(1, _NK):
                            zacc = zacc + xs[cl] * wz[cl][si]
                        zbuf[pl.ds(si * (_HIN * _ROWP) + r * _ROWP + jb, _L)] = zacc

            pl.loop(0, _HIN, unroll=2)(rowA2)

            def rowB(i):
                r = i + 1
                for jb in _JB_OUT:
                    a1 = jnp.zeros((_L,), jnp.float32)
                    for si, s in enumerate(_SHIFTS):
                        v = ybuf[pl.ds(si * (_HIN * _ROWP) + r * _ROWP
                                       + 1 + jb + s, _L)]
                        a1 = a1 + jnp.where(_col_mask(s, jb), v, 0.0)
                    a2 = jnp.zeros((_L,), jnp.float32)
                    for si, s in enumerate(_SHIFTS):
                        rr = r + s
                        ok = (rr >= 0) & (rr < _HIN)
                        rrc = jnp.clip(rr, 0, _HIN - 1)
                        v = zbuf[pl.ds(si * (_HIN * _ROWP) + rrc * _ROWP
                                       + 1 + jb, _L)]
                        a2 = a2 + jnp.where(ok, v, 0.0)
                    a3 = ubuf[pl.ds(r * _ROWP + 1 + jb, _L)]
                    ob1[pl.ds(base2 + i * _W + jb, _L)] = a1
                    ob2[pl.ds(base2 + i * _W + jb, _L)] = a2
                    ob3[pl.ds(base2 + i * _W + jb, _L)] = a3

            pl.loop(0, _H)(rowB)

        compute(xbuf)

        @pl.when(ul % 2 == 0)
        def _():
            pltpu.make_async_copy(ob1.at[pl.ds(base2, _OW)], o1_hbm.at[u], osem0).start()
            pltpu.make_async_copy(ob2.at[pl.ds(base2, _OW)], o2_hbm.at[u], osem0).start()
            pltpu.make_async_copy(ob3.at[pl.ds(base2, _OW)], o3_hbm.at[u], osem0).start()

        @pl.when(ul % 2 == 1)
        def _():
            pltpu.make_async_copy(ob1.at[pl.ds(base2, _OW)], o1_hbm.at[u], osem1).start()
            pltpu.make_async_copy(ob2.at[pl.ds(base2, _OW)], o2_hbm.at[u], osem1).start()
            pltpu.make_async_copy(ob3.at[pl.ds(base2, _OW)], o3_hbm.at[u], osem1).start()

    pl.loop(0, per_w)(unit_step)

    for ul_t in (per_w - 2, per_w - 1):
        u_t = wid * per_w + ul_t
        base2t = (ul_t % 2) * _OW
        sem = osem0 if ul_t % 2 == 0 else osem1
        pltpu.make_async_copy(ob1.at[pl.ds(base2t, _OW)], o1_hbm.at[u_t], sem).wait()
        pltpu.make_async_copy(ob2.at[pl.ds(base2t, _OW)], o2_hbm.at[u_t], sem).wait()
        pltpu.make_async_copy(ob3.at[pl.ds(base2t, _OW)], o3_hbm.at[u_t], sem).wait()


def _sc_call(x2, ph_t, w1, w2, id_t, w3):
    nunits = x2.shape[0]
    mesh = plsc.VectorSubcoreMesh(core_axis_name="c", subcore_axis_name="s",
                                  num_cores=2, num_subcores=16)
    oshape = jax.ShapeDtypeStruct((nunits, _OW), jnp.float32)
    kfn = pl.kernel(
        _body,
        mesh=mesh,
        compiler_params=pltpu.CompilerParams(use_tc_tiling_on_sc=False),
        out_type=[oshape, oshape, oshape],
        scratch_types=[
            pltpu.VMEM((2 * _XB,), jnp.float32),
            pltpu.VMEM((_NK * _HIN * _ROWP,), jnp.float32),
            pltpu.VMEM((_NK * _HIN * _ROWP,), jnp.float32),
            pltpu.VMEM((_HIN * _ROWP,), jnp.float32),
            pltpu.VMEM((2 * _OW,), jnp.float32),
            pltpu.VMEM((2 * _OW,), jnp.float32),
            pltpu.VMEM((2 * _OW,), jnp.float32),
            pltpu.VMEM((_NK * _CIN,), jnp.float32),
            pltpu.VMEM((_NK * _CIN,), jnp.float32),
            pltpu.VMEM((_NK * _COUT,), jnp.float32),
            pltpu.SemaphoreType.DMA,
            pltpu.SemaphoreType.DMA,
            pltpu.SemaphoreType.DMA,
            pltpu.SemaphoreType.DMA,
        ],
    )
    return kfn(x2, ph_t, w1, w2, id_t, w3)


def kernel(x, pad_hv, idx_identit, idx_out, w1, w2, w3, b, hout, wout):
    B, c_in, Hin, Win = x.shape
    c_out = idx_identit.shape[0]
    x2 = x.reshape(B * c_out, _NK * Hin * Win)
    ph_f = pad_hv.T.reshape(-1).astype(jnp.float32)
    id_f = idx_identit.T.reshape(-1).astype(jnp.float32)
    o1, o2, o3 = _sc_call(x2, ph_f, w1, w2, id_f, w3)
    H, W = Hin - 2 * _EXTRA, Win - 2 * _EXTRA
    return (o1.reshape(B, c_out, H, W),
            o2.reshape(B, c_out, H, W),
            o3.reshape(B, c_out, H, W))


# speedup vs baseline: 17.0983x; 1.0546x over previous
import math

import jax
import jax.numpy as jnp
from jax import lax
from jax.experimental import pallas as pl
from jax.experimental.pallas import tpu as pltpu
from jax.experimental.pallas import tpu_sc as plsc

_BIG_K = 13
_SMALL_K = 3
_NK = math.ceil(_BIG_K / _SMALL_K)
_PAD = _SMALL_K - 1
_MID = _BIG_K // 2
_SHIFTS = tuple(_MID - i * _SMALL_K - _PAD for i in range(_NK))
_EXTRA = _PAD - _SMALL_K // 2

_L = 16
_HIN = 58
_WIN = 58
_H = 56
_W = 56
_ROWP = 64
_CROW = _HIN * _WIN
_XW = _NK * _CROW
_XCP = _XW + 12
_XB = _XW + 28
_OW = _H * _W
_G = 4
_CIN = 480
_COUT = 96
_JB_IN = (0, 16, 32, 42)
_JB_OUT = (0, 16, 32, 40)
_NW = 32


def _iota():
    return lax.iota(jnp.int32, _L)


def _splat_lane(vec, lane):
    return jnp.take_along_axis(
        vec, jnp.full((_L,), lane, jnp.int32), axis=0,
        mode=lax.GatherScatterMode.PROMISE_IN_BOUNDS)


def _col_mask(s, jbase):
    q = 1 + _iota() + jbase + s
    return (q >= 0) & (q < _WIN)


def _body(x_hbm, ph_hbm, w1_hbm, w2_hbm, id_hbm, w3_hbm,
          o1_hbm, o2_hbm, o3_hbm,
          xbuf, ybuf, zbuf, ubuf, ob1, ob2, ob3,
          ws1, ws2, ws3,
          dsem0, dsem1, osem0, osem1):
    nunits = o1_hbm.shape[0]
    wid = lax.axis_index("s") * 2 + lax.axis_index("c")
    per_w = nunits // _NW

    _PH, _W1, _W2, _W3, _ID = 0, 3840, 5760, 7680, 8064
    pltpu.sync_copy(ph_hbm, zbuf.at[pl.ds(_PH, 2 * _G * _CIN)])
    pltpu.sync_copy(w1_hbm, zbuf.at[pl.ds(_W1, _G * _CIN)])
    pltpu.sync_copy(w2_hbm, zbuf.at[pl.ds(_W2, _G * _CIN)])
    pltpu.sync_copy(w3_hbm, zbuf.at[pl.ds(_W3, _G * _COUT)])
    pltpu.sync_copy(id_hbm, zbuf.at[pl.ds(_ID, _G * _COUT)])

    for cv in range(_CIN // _L):
        c0 = cv * _L
        for si, s in enumerate(_SHIFTS):
            acc1 = jnp.zeros((_L,), jnp.float32)
            acc2 = jnp.zeros((_L,), jnp.float32)
            for g in range(_G):
                ph1 = zbuf[pl.ds(_PH + g * _CIN + c0, _L)]
                ph2 = zbuf[pl.ds(_PH + (_G + g) * _CIN + c0, _L)]
                wa = zbuf[pl.ds(_W1 + g * _CIN + c0, _L)]
                wb = zbuf[pl.ds(_W2 + g * _CIN + c0, _L)]
                acc1 = acc1 + jnp.where(ph1 == float(s), wa, 0.0)
                acc2 = acc2 + jnp.where(ph2 == float(s), wb, 0.0)
            ws1[pl.ds(si * _CIN + c0, _L)] = acc1
            ws2[pl.ds(si * _CIN + c0, _L)] = acc2
    for cov in range(_COUT // _L):
        co0 = cov * _L
        covec = jnp.full((_L,), co0, jnp.int32) + _iota()
        for k in range(_NK):
            acc3 = jnp.zeros((_L,), jnp.float32)
            for g in range(_G):
                idg = zbuf[pl.ds(_ID + g * _COUT + co0, _L)]
                wg = zbuf[pl.ds(_W3 + g * _COUT + co0, _L)]
                acc3 = acc3 + jnp.where(idg == (covec * _NK + k).astype(jnp.float32),
                                        wg, 0.0)
            ws3[pl.ds(k * _COUT + co0, _L)] = acc3

    def unit_of(ul):
        return wid * per_w + ul

    def in_cp(uu, slot_base, sem):
        off = (uu % 4) * 4
        start = pl.multiple_of(uu * _XW - off, 16)
        return pltpu.make_async_copy(
            x_hbm.at[pl.ds(start, _XCP)],
            xbuf.at[pl.ds(slot_base, _XCP)], sem)

    in_cp(unit_of(0), 0, dsem0).start()

    def unit_step(ul):
        u = unit_of(ul)
        base = (ul % 2) * _XB + (u % 4) * 4
        base2 = (ul % 2) * _OW

        @pl.when((ul >= 2) & (ul % 2 == 0))
        def _():
            pltpu.make_async_copy(ob1.at[pl.ds(base2, _OW)], o1_hbm.at[u - 2], osem0).wait()
            pltpu.make_async_copy(ob2.at[pl.ds(base2, _OW)], o2_hbm.at[u - 2], osem0).wait()
            pltpu.make_async_copy(ob3.at[pl.ds(base2, _OW)], o3_hbm.at[u - 2], osem0).wait()

        @pl.when((ul >= 2) & (ul % 2 == 1))
        def _():
            pltpu.make_async_copy(ob1.at[pl.ds(base2, _OW)], o1_hbm.at[u - 2], osem1).wait()
            pltpu.make_async_copy(ob2.at[pl.ds(base2, _OW)], o2_hbm.at[u - 2], osem1).wait()
            pltpu.make_async_copy(ob3.at[pl.ds(base2, _OW)], o3_hbm.at[u - 2], osem1).wait()

        @pl.when(ul % 2 == 0)
        def _():
            in_cp(u, 0, dsem0).wait()

        @pl.when(ul % 2 == 1)
        def _():
            in_cp(u, _XB, dsem1).wait()

        @pl.when((ul + 1 < per_w) & (ul % 2 == 0))
        def _():
            in_cp(unit_of(ul + 1), _XB, dsem1).start()

        @pl.when((ul + 1 < per_w) & (ul % 2 == 1))
        def _():
            in_cp(unit_of(ul + 1), 0, dsem0).start()

        co = u % _COUT
        c0 = co * _NK

        wy = [[None] * _NK for _ in range(_NK)]
        wz = [[None] * _NK for _ in range(_NK)]
        wu = [None] * _NK
        for si in range(_NK):
            v1 = ws1[pl.ds(si * _CIN + c0, _L)]
            v2 = ws2[pl.ds(si * _CIN + c0, _L)]
            for cl in range(_NK):
                wy[cl][si] = _splat_lane(v1, cl)
                wz[cl][si] = _splat_lane(v2, cl)
        lane3 = co % _L
        for cl in range(_NK):
            v3 = ws3[pl.ds(cl * _COUT + (co // _L) * _L, _L)]
            wu[cl] = _splat_lane(v3, lane3)

        def compute(xref):
            def rowA1(r):
                for jb in _JB_IN:
                    xs = [xref[pl.ds(base + cl * _CROW + r * _WIN + jb, _L)]
                          for cl in range(_NK)]
                    uacc = xs[0] * wu[0]
                    for cl in range(1, _NK):
                        uacc = uacc + xs[cl] * wu[cl]
                    ubuf[pl.ds(r * _ROWP + jb, _L)] = uacc
                    for si in range(_NK):
                        yacc = xs[0] * wy[0][si]
                        for cl in range(1, _NK):
                            yacc = yacc + xs[cl] * wy[cl][si]
                        ybuf[pl.ds(si * (_HIN * _ROWP) + r * _ROWP + jb, _L)] = yacc

            pl.loop(0, _HIN, unroll=2)(rowA1)

            def rowA2(r):
                for jb in _JB_IN:
                    xs = [xref[pl.ds(base + cl * _CROW + r * _WIN + jb, _L)]
                          for cl in range(_NK)]
                    for si in range(_NK):
                        zacc = xs[0] * wz[0][si]
                        for cl in range(1, _NK):
                            zacc = zacc + xs[cl] * wz[cl][si]
                        zbuf[pl.ds(si * (_HIN * _ROWP) + r * _ROWP + jb, _L)] = zacc

            pl.loop(0, _HIN, unroll=2)(rowA2)

            def rowB(i):
                r = i + 1
                for jb in _JB_OUT:
                    a1 = jnp.zeros((_L,), jnp.float32)
                    for si, s in enumerate(_SHIFTS):
                        v = ybuf[pl.ds(si * (_HIN * _ROWP) + r * _ROWP
                                       + 1 + jb + s, _L)]
                        a1 = a1 + jnp.where(_col_mask(s, jb), v, 0.0)
                    a2 = jnp.zeros((_L,), jnp.float32)
                    for si, s in enumerate(_SHIFTS):
                        rr = r + s
                        ok = (rr >= 0) & (rr < _HIN)
                        rrc = jnp.clip(rr, 0, _HIN - 1)
                        v = zbuf[pl.ds(si * (_HIN * _ROWP) + rrc * _ROWP
                                       + 1 + jb, _L)]
                        a2 = a2 + jnp.where(ok, v, 0.0)
                    a3 = ubuf[pl.ds(r * _ROWP + 1 + jb, _L)]
                    ob1[pl.ds(base2 + i * _W + jb, _L)] = a1
                    ob2[pl.ds(base2 + i * _W + jb, _L)] = a2
                    ob3[pl.ds(base2 + i * _W + jb, _L)] = a3

            pl.loop(0, _H)(rowB)

        compute(xbuf)

        @pl.when(ul % 2 == 0)
        def _():
            pltpu.make_async_copy(ob1.at[pl.ds(base2, _OW)], o1_hbm.at[u], osem0).start()
            pltpu.make_async_copy(ob2.at[pl.ds(base2, _OW)], o2_hbm.at[u], osem0).start()
            pltpu.make_async_copy(ob3.at[pl.ds(base2, _OW)], o3_hbm.at[u], osem0).start()

        @pl.when(ul % 2 == 1)
        def _():
            pltpu.make_async_copy(ob1.at[pl.ds(base2, _OW)], o1_hbm.at[u], osem1).start()
            pltpu.make_async_copy(ob2.at[pl.ds(base2, _OW)], o2_hbm.at[u], osem1).start()
            pltpu.make_async_copy(ob3.at[pl.ds(base2, _OW)], o3_hbm.at[u], osem1).start()

    pl.loop(0, per_w)(unit_step)

    for ul_t in (per_w - 2, per_w - 1):
        u_t = wid * per_w + ul_t
        base2t = (ul_t % 2) * _OW
        sem = osem0 if ul_t % 2 == 0 else osem1
        pltpu.make_async_copy(ob1.at[pl.ds(base2t, _OW)], o1_hbm.at[u_t], sem).wait()
        pltpu.make_async_copy(ob2.at[pl.ds(base2t, _OW)], o2_hbm.at[u_t], sem).wait()
        pltpu.make_async_copy(ob3.at[pl.ds(base2t, _OW)], o3_hbm.at[u_t], sem).wait()


def _sc_call(x2, ph_t, w1, w2, id_t, w3, nunits):
    mesh = plsc.VectorSubcoreMesh(core_axis_name="c", subcore_axis_name="s",
                                  num_cores=2, num_subcores=16)
    oshape = jax.ShapeDtypeStruct((nunits, _OW), jnp.float32)
    kfn = pl.kernel(
        _body,
        mesh=mesh,
        compiler_params=pltpu.CompilerParams(use_tc_tiling_on_sc=False),
        out_type=[oshape, oshape, oshape],
        scratch_types=[
            pltpu.VMEM((2 * _XB,), jnp.float32),
            pltpu.VMEM((_NK * _HIN * _ROWP,), jnp.float32),
            pltpu.VMEM((_NK * _HIN * _ROWP,), jnp.float32),
            pltpu.VMEM((_HIN * _ROWP,), jnp.float32),
            pltpu.VMEM((2 * _OW,), jnp.float32),
            pltpu.VMEM((2 * _OW,), jnp.float32),
            pltpu.VMEM((2 * _OW,), jnp.float32),
            pltpu.VMEM((_NK * _CIN,), jnp.float32),
            pltpu.VMEM((_NK * _CIN,), jnp.float32),
            pltpu.VMEM((_NK * _COUT,), jnp.float32),
            pltpu.SemaphoreType.DMA,
            pltpu.SemaphoreType.DMA,
            pltpu.SemaphoreType.DMA,
            pltpu.SemaphoreType.DMA,
        ],
    )
    return kfn(x2, ph_t, w1, w2, id_t, w3)


def kernel(x, pad_hv, idx_identit, idx_out, w1, w2, w3, b, hout, wout):
    B, c_in, Hin, Win = x.shape
    c_out = idx_identit.shape[0]
    x2 = x.reshape(-1)
    ph_f = pad_hv.T.reshape(-1).astype(jnp.float32)
    id_f = idx_identit.T.reshape(-1).astype(jnp.float32)
    o1, o2, o3 = _sc_call(x2, ph_f, w1, w2, id_f, w3, B * c_out)
    H, W = Hin - 2 * _EXTRA, Win - 2 * _EXTRA
    return (o1.reshape(B, c_out, H, W),
            o2.reshape(B, c_out, H, W),
            o3.reshape(B, c_out, H, W))
